# Initial kernel scaffold; baseline (speedup 1.0000x reference)
import jax
import jax.numpy as jnp
from jax.experimental import pallas as pl


def kernel(medicine_it, m_embeddings, pretrained_model, W, bias):
    raise NotImplementedError("write your pallas kernel here")


# speedup vs baseline: 12.5061x; 12.5061x over previous
import functools

import jax
import jax.numpy as jnp
from jax import lax
from jax.experimental import pallas as pl
from jax.experimental.pallas import tpu as pltpu
from jax.experimental.pallas import tpu_sc as plsc

D = 128
NW = 32
C = 112
NSUB = 14
CHUNK = NSUB * C
TOT = NW * CHUNK


def _sc_gather_sum(idx_hbm, m_hbm, p_hbm, out_hbm, idx_v, buf_m, buf_p, row_v, sem):
    wid = lax.axis_index("s") * 2 + lax.axis_index("c")
    pltpu.sync_copy(idx_hbm.at[wid], idx_v)
    accs = tuple(jnp.zeros((16,), jnp.float32) for _ in range(16))
    for j in range(NSUB):
        pltpu.async_copy(m_hbm.at[idx_v.at[j]], buf_m, sem).wait()
        pltpu.async_copy(p_hbm.at[idx_v.at[j]], buf_p, sem).wait()

        def body(i, a):
            new = [a[k] + buf_m[i, pl.ds(16 * k, 16)] for k in range(8)]
            new += [a[8 + k] + buf_p[i, pl.ds(16 * k, 16)] for k in range(8)]
            return tuple(new)

        accs = lax.fori_loop(0, C, body, accs)
    for k in range(16):
        row_v[pl.ds(16 * k, 16)] = accs[k]
    pltpu.sync_copy(row_v, out_hbm.at[wid])


def _gather_sums(idx3, m_emb, pre):
    mesh = plsc.VectorSubcoreMesh(core_axis_name="c", subcore_axis_name="s")
    f = pl.kernel(
        _sc_gather_sum,
        mesh=mesh,
        out_type=jax.ShapeDtypeStruct((NW, 2 * D), jnp.float32),
        scratch_types=[
            pltpu.VMEM((NSUB, C), jnp.int32),
            pltpu.VMEM((C, D), jnp.float32),
            pltpu.VMEM((C, D), jnp.float32),
            pltpu.VMEM((2 * D,), jnp.float32),
            pltpu.SemaphoreType.DMA,
        ],
    )
    return f(idx3, m_emb, pre)


def _combine_body(parts_ref, w_ref, bias_ref, m0_ref, p0_ref, out_ref, *, n, pad):
    s = jnp.sum(parts_ref[:, :], axis=0, keepdims=True)
    s_m = s[:, :D] - jnp.float32(pad) * m0_ref[:, :]
    s_p = s[:, D:] - jnp.float32(pad) * p0_ref[:, :]
    y = lax.dot_general(s_m, w_ref[:, :], (((1,), (1,)), ((), ())),
                        preferred_element_type=jnp.float32)
    out_ref[:, :] = s_p + y + jnp.float32(n) * bias_ref[:, :]


def kernel(medicine_it, m_embeddings, pretrained_model, W, bias):
    n = medicine_it.shape[0]
    pad = TOT - n
    idx = jnp.concatenate(
        [medicine_it.astype(jnp.int32), jnp.zeros((pad,), jnp.int32)])
    idx3 = idx.reshape(NW, NSUB, C)
    parts = _gather_sums(idx3, m_embeddings, pretrained_model)
    out = pl.pallas_call(
        functools.partial(_combine_body, n=n, pad=pad),
        out_shape=jax.ShapeDtypeStruct((1, D), jnp.float32),
    )(parts, W, bias.reshape(1, D), m_embeddings[0:1], pretrained_model[0:1])
    return out.reshape(1, 1, D)


# speedup vs baseline: 19.4382x; 1.5543x over previous
import functools

import jax
import jax.numpy as jnp
from jax import lax
from jax.experimental import pallas as pl
from jax.experimental.pallas import tpu as pltpu
from jax.experimental.pallas import tpu_sc as plsc

D = 128
NW = 32
C = 112
NSUB = 14
CHUNK = NSUB * C
TOT = NW * CHUNK


def _sc_gather_sum(idx_hbm, m_hbm, p_hbm, out_hbm, idx_v,
                   bm0, bm1, bp0, bp1, row_v, sem0, sem1):
    wid = lax.axis_index("s") * 2 + lax.axis_index("c")
    pltpu.sync_copy(idx_hbm.at[wid], idx_v)
    bm, bp, sems = (bm0, bm1), (bp0, bp1), (sem0, sem1)

    def start(j):
        s = j % 2
        return (pltpu.async_copy(m_hbm.at[idx_v.at[j]], bm[s], sems[s]),
                pltpu.async_copy(p_hbm.at[idx_v.at[j]], bp[s], sems[s]))

    accs = tuple(jnp.zeros((16,), jnp.float32) for _ in range(16))
    pending = start(0)
    for j in range(NSUB):
        nxt = start(j + 1) if j + 1 < NSUB else None
        for d in pending:
            d.wait()
        s = j % 2

        def body(i, a, _bm=bm[s], _bp=bp[s]):
            new = [a[k] + _bm[i, pl.ds(16 * k, 16)] for k in range(8)]
            new += [a[8 + k] + _bp[i, pl.ds(16 * k, 16)] for k in range(8)]
            return tuple(new)

        accs = lax.fori_loop(0, C, body, accs)
        pending = nxt
    for k in range(16):
        row_v[pl.ds(16 * k, 16)] = accs[k]
    pltpu.sync_copy(row_v, out_hbm.at[wid])


def _gather_sums(idx3, m_emb, pre):
    mesh = plsc.VectorSubcoreMesh(core_axis_name="c", subcore_axis_name="s")
    f = pl.kernel(
        _sc_gather_sum,
        mesh=mesh,
        out_type=jax.ShapeDtypeStruct((NW, 2 * D), jnp.float32),
        scratch_types=[
            pltpu.VMEM((NSUB, C), jnp.int32),
            pltpu.VMEM((C, D), jnp.float32),
            pltpu.VMEM((C, D), jnp.float32),
            pltpu.VMEM((C, D), jnp.float32),
            pltpu.VMEM((C, D), jnp.float32),
            pltpu.VMEM((2 * D,), jnp.float32),
            pltpu.SemaphoreType.DMA,
            pltpu.SemaphoreType.DMA,
        ],
    )
    return f(idx3, m_emb, pre)


def _combine_body(parts_ref, w_ref, bias_ref, m0_ref, p0_ref, out_ref, *, n, pad):
    s = jnp.sum(parts_ref[:, :], axis=0, keepdims=True)
    s_m = s[:, :D] - jnp.float32(pad) * m0_ref[:, :]
    s_p = s[:, D:] - jnp.float32(pad) * p0_ref[:, :]
    y = lax.dot_general(s_m, w_ref[:, :], (((1,), (1,)), ((), ())),
                        preferred_element_type=jnp.float32)
    out_ref[:, :] = s_p + y + jnp.float32(n) * bias_ref[:, :]


def kernel(medicine_it, m_embeddings, pretrained_model, W, bias):
    n = medicine_it.shape[0]
    pad = TOT - n
    idx = jnp.concatenate(
        [medicine_it.astype(jnp.int32), jnp.zeros((pad,), jnp.int32)])
    idx3 = idx.reshape(NW, NSUB, C)
    parts = _gather_sums(idx3, m_embeddings, pretrained_model)
    out = pl.pallas_call(
        functools.partial(_combine_body, n=n, pad=pad),
        out_shape=jax.ShapeDtypeStruct((1, D), jnp.float32),
    )(parts, W, bias.reshape(1, D), m_embeddings[0:1], pretrained_model[0:1])
    return out.reshape(1, 1, D)
